# baseline (speedup 1.0000x reference)
import functools

import jax
import jax.numpy as jnp
from jax import lax
from jax.experimental import pallas as pl
from jax.experimental.pallas import tpu as pltpu
from jax.experimental.pallas import tpu_sc as plsc

_ACTION_DIM = 1024
_ACTION = (7, 42, 123)
_LANES = 16
_NUM_WORKERS = 32
_SC_BUF_ROWS = 64
_TC_BUF_ROWS = 1024
_SPLIT = 10240


def _sc_fill_body(out_hbm, buf, sem):
    wid = lax.axis_index("s") * 2 + lax.axis_index("c")
    lane = lax.iota(jnp.int32, _LANES)

    def _fill_row(r, carry):
        for g in range(_ACTION_DIM // _LANES):
            base_col = g * _LANES
            v = jnp.zeros((_LANES,), jnp.float32)
            for a in _ACTION:
                if base_col <= a < base_col + _LANES:
                    v = jnp.where(lane == (a - base_col), 1.0, v)
            buf[r, pl.ds(base_col, _LANES)] = v
        return carry

    lax.fori_loop(0, _SC_BUF_ROWS, _fill_row, 0)

    rows_per_worker = _SPLIT // _NUM_WORKERS
    base = wid * rows_per_worker
    copies = []
    for i in range(rows_per_worker // _SC_BUF_ROWS):
        dst = out_hbm.at[pl.ds(base + i * _SC_BUF_ROWS, _SC_BUF_ROWS), :]
        copies.append(pltpu.async_copy(buf, dst, sem))
    for c in copies:
        c.wait()


def _tc_fill_body(out_ref, buf, sem):
    col = jax.lax.broadcasted_iota(jnp.int32, (_TC_BUF_ROWS, _ACTION_DIM), 1)
    mask = (col == _ACTION[0]) | (col == _ACTION[1]) | (col == _ACTION[2])
    buf[...] = mask.astype(jnp.float32)
    n_blocks = out_ref.shape[0] // _TC_BUF_ROWS
    for i in range(n_blocks):
        dst = out_ref.at[pl.ds(i * _TC_BUF_ROWS, _TC_BUF_ROWS), :]
        pltpu.make_async_copy(buf, dst, sem).start()
    for i in range(n_blocks):
        dst = out_ref.at[pl.ds(i * _TC_BUF_ROWS, _TC_BUF_ROWS), :]
        pltpu.make_async_copy(buf, dst, sem).wait()


def kernel(hidden, obs, done):
    n_rows = obs.shape[1]

    mesh = plsc.VectorSubcoreMesh(core_axis_name="c", subcore_axis_name="s")
    sc_fill = functools.partial(
        pl.kernel,
        mesh=mesh,
        out_type=jax.ShapeDtypeStruct((_SPLIT, _ACTION_DIM), jnp.float32),
        scratch_types=[
            pltpu.VMEM((_SC_BUF_ROWS, _ACTION_DIM), jnp.float32),
            pltpu.SemaphoreType.DMA,
        ],
    )(_sc_fill_body)
    probs_top = sc_fill()

    probs_bottom = pl.pallas_call(
        _tc_fill_body,
        out_specs=pl.BlockSpec(memory_space=pltpu.MemorySpace.HBM),
        out_shape=jax.ShapeDtypeStruct((n_rows - _SPLIT, _ACTION_DIM), jnp.float32),
        scratch_shapes=[
            pltpu.VMEM((_TC_BUF_ROWS, _ACTION_DIM), jnp.float32),
            pltpu.SemaphoreType.DMA,
        ],
    )()

    probs = jnp.concatenate([probs_top, probs_bottom], axis=0)
    critic = jnp.asarray(0)
    return (hidden, probs, critic)


# speedup vs baseline: 1.5881x; 1.5881x over previous
import functools

import jax
import jax.numpy as jnp
from jax import lax
from jax.experimental import pallas as pl
from jax.experimental.pallas import tpu as pltpu
from jax.experimental.pallas import tpu_sc as plsc

_ACTION_DIM = 1024
_ACTION = (7, 42, 123)
_LANES = 16
_NUM_WORKERS = 32
_SC_BUF_ROWS = 16


def _sc_fill_body(out_hbm, buf, sem):
    wid = lax.axis_index("s") * 2 + lax.axis_index("c")
    lane = lax.iota(jnp.int32, _LANES)

    def _fill_row(r, carry):
        for g in range(_ACTION_DIM // _LANES):
            base_col = g * _LANES
            v = jnp.zeros((_LANES,), jnp.float32)
            for a in _ACTION:
                if base_col <= a < base_col + _LANES:
                    v = jnp.where(lane == (a - base_col), 1.0, v)
            buf[r, pl.ds(base_col, _LANES)] = v
        return carry

    lax.fori_loop(0, _SC_BUF_ROWS, _fill_row, 0)

    n_rows = out_hbm.shape[0]
    rows_per_worker = n_rows // _NUM_WORKERS
    base = wid * rows_per_worker
    copies = []
    for i in range(rows_per_worker // _SC_BUF_ROWS):
        dst = out_hbm.at[pl.ds(base + i * _SC_BUF_ROWS, _SC_BUF_ROWS), :]
        copies.append(pltpu.async_copy(buf, dst, sem))
    for c in copies:
        c.wait()


def kernel(hidden, obs, done):
    n_rows = obs.shape[1]
    mesh = plsc.VectorSubcoreMesh(core_axis_name="c", subcore_axis_name="s")
    sc_fill = functools.partial(
        pl.kernel,
        mesh=mesh,
        out_type=jax.ShapeDtypeStruct((n_rows, _ACTION_DIM), jnp.float32),
        scratch_types=[
            pltpu.VMEM((_SC_BUF_ROWS, _ACTION_DIM), jnp.float32),
            pltpu.SemaphoreType.DMA,
        ],
    )(_sc_fill_body)
    probs = sc_fill()
    critic = jnp.asarray(0)
    return (hidden, probs, critic)
